# baseline (speedup 1.0000x reference)
import functools

import jax
import jax.numpy as jnp
from jax import lax
from jax.experimental import pallas as pl
from jax.experimental.pallas import tpu as pltpu
from jax.experimental.pallas import tpu_sc as plsc

N_NODES = 10000
D = 128
NC, NS, L = 2, 16, 16
CHUNK = 64
NCHUNK = 320
NPAD = 10240
ZROWS = NPAD // NS
NBUF = 4
IBUF = 2 * NBUF
NCH0 = 272
NCH1 = NCHUNK - NCH0


def _sc_agg_body(with_deg, *refs):
  if with_deg:
    (h_hbm, src_hbm, dst_hbm, z2_hbm, z1_hbm, acc_out, deg_out,
     acc_sh, deg_sh, src_i, dst_i, rows_v, ones_v, *sems) = refs
  else:
    (h_hbm, src_hbm, dst_hbm, z2_hbm, acc_out,
     acc_sh, src_i, dst_i, rows_v, *sems) = refs
  gsem = sems[:NBUF]
  ssem = sems[NBUF:2 * NBUF]
  isem = sems[2 * NBUF:]
  c = lax.axis_index("c")
  s = lax.axis_index("s")
  nch = jnp.where(c == 0, NCH0, NCH1)
  base = jnp.where(c == 0, s * NCH0, NS * NCH0 + s * NCH1) * CHUNK

  def idx_load(k, slot):
    pltpu.async_copy(src_hbm.at[pl.ds(base + k * CHUNK, CHUNK)],
                     src_i.at[slot], isem[slot])
    pltpu.async_copy(dst_hbm.at[pl.ds(base + k * CHUNK, CHUNK)],
                     dst_i.at[slot], isem[slot])

  def idx_wait(k, slot):
    pltpu.make_async_copy(src_hbm.at[pl.ds(base + k * CHUNK, CHUNK)],
                          src_i.at[slot], isem[slot]).wait()
    pltpu.make_async_copy(dst_hbm.at[pl.ds(base + k * CHUNK, CHUNK)],
                          dst_i.at[slot], isem[slot]).wait()

  pltpu.sync_copy(z2_hbm, acc_sh.at[pl.ds(s * ZROWS, ZROWS)])
  if with_deg:
    pltpu.sync_copy(z1_hbm, deg_sh.at[pl.ds(s * ZROWS, ZROWS)])
    for i in range(CHUNK // L):
      ones_v[pl.ds(i * L, L)] = jnp.ones((L,), jnp.float32)

  for k in range(IBUF):
    idx_load(k, k)
  for b in range(NBUF):
    idx_wait(b, b)
    pltpu.async_copy(h_hbm.at[src_i.at[b]], rows_v.at[b], gsem[b])
  plsc.subcore_barrier()

  def group(g, carry):
    for u in range(IBUF):
      j = g * IBUF + u
      b = u % NBUF
      s2 = (u + NBUF) % IBUF
      pltpu.make_async_copy(h_hbm.at[src_i.at[u]], rows_v.at[b],
                            gsem[b]).wait()
      pltpu.async_copy(rows_v.at[b], acc_sh.at[dst_i.at[u]], ssem[b],
                       add=True)
      if with_deg:
        pltpu.sync_copy(ones_v, deg_sh.at[dst_i.at[u]], add=True)
      pltpu.make_async_copy(rows_v.at[b], acc_sh.at[dst_i.at[u]],
                            ssem[b]).wait()

      @pl.when(j + IBUF < nch)
      def _():
        idx_load(j + IBUF, u)

      @pl.when(j + NBUF < nch)
      def _():
        idx_wait(j + NBUF, s2)
        pltpu.async_copy(h_hbm.at[src_i.at[s2]], rows_v.at[b], gsem[b])
    return carry

  lax.fori_loop(0, nch // IBUF, group, 0)
  plsc.subcore_barrier()

  pltpu.sync_copy(acc_sh.at[pl.ds(s * ZROWS, ZROWS)],
                  acc_out.at[c, pl.ds(s * ZROWS, ZROWS)])
  if with_deg:
    pltpu.sync_copy(deg_sh.at[pl.ds(s * ZROWS, ZROWS)],
                    deg_out.at[pl.ds(c * NPAD + s * ZROWS, ZROWS)])


def _make_sc_agg(with_deg):
  mesh = plsc.VectorSubcoreMesh(core_axis_name="c", subcore_axis_name="s",
                                num_cores=NC, num_subcores=NS)
  out_type = [jax.ShapeDtypeStruct((NC, NPAD, D), jnp.float32)]
  scratch = [
      pltpu.VMEM_SHARED((NPAD, D), jnp.float32),
  ]
  if with_deg:
    out_type.append(jax.ShapeDtypeStruct((NC * NPAD,), jnp.float32))
    scratch.append(pltpu.VMEM_SHARED((NPAD,), jnp.float32))
  scratch += [
      pltpu.VMEM((IBUF, CHUNK), jnp.int32),
      pltpu.VMEM((IBUF, CHUNK), jnp.int32),
      pltpu.VMEM((NBUF, CHUNK, D), jnp.float32),
  ]
  if with_deg:
    scratch.append(pltpu.VMEM((CHUNK,), jnp.float32))
  scratch += [pltpu.SemaphoreType.DMA] * (2 * NBUF + IBUF)
  return pl.kernel(
      functools.partial(_sc_agg_body, with_deg),
      out_type=tuple(out_type),
      mesh=mesh,
      scratch_types=tuple(scratch),
  )


_sc_agg_deg = _make_sc_agg(True)
_sc_agg = _make_sc_agg(False)


def _mm_body(x_ref, w_ref, b_ref, o_ref):
  o_ref[...] = (
      jnp.dot(x_ref[...], w_ref[...], preferred_element_type=jnp.float32)
      + b_ref[...]
  )


def _mm(x, w, b):
  m = x.shape[0]
  bm = 1000
  return pl.pallas_call(
      _mm_body,
      grid=(m // bm,),
      in_specs=[
          pl.BlockSpec((bm, D), lambda i: (i, 0)),
          pl.BlockSpec((D, D), lambda i: (0, 0)),
          pl.BlockSpec((1, D), lambda i: (0, 0)),
      ],
      out_specs=pl.BlockSpec((bm, D), lambda i: (i, 0)),
      out_shape=jax.ShapeDtypeStruct((m, D), jnp.float32),
  )(x, w, b)


def _scale_mm_body(p_ref, d_ref, w_ref, b_ref, o_ref):
  a = (p_ref[0] + p_ref[1]) * (d_ref[0] + d_ref[1])
  o_ref[...] = (
      jnp.dot(a, w_ref[...], preferred_element_type=jnp.float32) + b_ref[...]
  )


def _scale_mm(parts, degcol, w, b):
  m = parts.shape[1]
  bm = 1024
  return pl.pallas_call(
      _scale_mm_body,
      grid=(m // bm,),
      in_specs=[
          pl.BlockSpec((NC, bm, D), lambda i: (0, i, 0)),
          pl.BlockSpec((NC, bm, 1), lambda i: (0, i, 0)),
          pl.BlockSpec((D, D), lambda i: (0, 0)),
          pl.BlockSpec((1, D), lambda i: (0, 0)),
      ],
      out_specs=pl.BlockSpec((bm, D), lambda i: (i, 0)),
      out_shape=jax.ShapeDtypeStruct((m, D), jnp.float32),
  )(parts, degcol, w, b)


def kernel(x, edge_index, W1, b1, W2, b2, W3, b3):
  src = edge_index[0].astype(jnp.int32)
  dst = edge_index[1].astype(jnp.int32)
  e = src.shape[0]
  pad = NS * NCHUNK * CHUNK - e
  src3 = jnp.concatenate([src, jnp.zeros((pad,), jnp.int32)])
  dst3 = jnp.concatenate([dst, jnp.full((pad,), N_NODES, jnp.int32)])
  z2 = jnp.zeros((ZROWS, D), jnp.float32)
  z1 = jnp.zeros((ZROWS,), jnp.float32)
  b1r = b1.reshape(1, D)
  b2r = b2.reshape(1, D)
  n_cls = W3.shape[1]
  w3p = jnp.zeros((D, D), jnp.float32).at[:, :n_cls].set(W3)
  b3p = jnp.zeros((1, D), jnp.float32).at[0, :n_cls].set(b3)

  h1 = _mm(x, W1, b1r)
  acc1, degp = _sc_agg_deg(h1, src3, dst3, z2, z1)
  degcol = degp.reshape(NC, NPAD, 1)
  h2 = _scale_mm(acc1, degcol, W2, b2r)
  (acc2,) = _sc_agg(h2, src3, dst3, z2)
  outp = _scale_mm(acc2, degcol, w3p, b3p)
  return outp[:N_NODES, :n_cls]


# speedup vs baseline: 1.0554x; 1.0554x over previous
import functools

import jax
import jax.numpy as jnp
from jax import lax
from jax.experimental import pallas as pl
from jax.experimental.pallas import tpu as pltpu
from jax.experimental.pallas import tpu_sc as plsc

N_NODES = 10000
D = 128
NC, NS, L = 2, 16, 16
CHUNK = 64
NCHUNK = 320
NPAD = 10240
ZROWS = NPAD // NS
NBUF = 4
IBUF = 2 * NBUF
NCH0 = 288
NCH1 = NCHUNK - NCH0


def _sc_agg_body(with_deg, *refs):
  if with_deg:
    (h_hbm, src_hbm, dst_hbm, z2_hbm, z1_hbm, acc_out, deg_out,
     acc_sh, deg_sh, src_i, dst_i, rows_v, ones_v, *sems) = refs
  else:
    (h_hbm, src_hbm, dst_hbm, z2_hbm, acc_out,
     acc_sh, src_i, dst_i, rows_v, *sems) = refs
  gsem = sems[:NBUF]
  ssem = sems[NBUF:2 * NBUF]
  isem = sems[2 * NBUF:]
  c = lax.axis_index("c")
  s = lax.axis_index("s")
  nch = jnp.where(c == 0, NCH0, NCH1)
  base = jnp.where(c == 0, s * NCH0, NS * NCH0 + s * NCH1) * CHUNK

  def idx_load(k, slot):
    pltpu.async_copy(src_hbm.at[pl.ds(base + k * CHUNK, CHUNK)],
                     src_i.at[slot], isem[slot])
    pltpu.async_copy(dst_hbm.at[pl.ds(base + k * CHUNK, CHUNK)],
                     dst_i.at[slot], isem[slot])

  def idx_wait(k, slot):
    pltpu.make_async_copy(src_hbm.at[pl.ds(base + k * CHUNK, CHUNK)],
                          src_i.at[slot], isem[slot]).wait()
    pltpu.make_async_copy(dst_hbm.at[pl.ds(base + k * CHUNK, CHUNK)],
                          dst_i.at[slot], isem[slot]).wait()

  pltpu.sync_copy(z2_hbm, acc_sh.at[pl.ds(s * ZROWS, ZROWS)])
  if with_deg:
    pltpu.sync_copy(z1_hbm, deg_sh.at[pl.ds(s * ZROWS, ZROWS)])
    for i in range(CHUNK // L):
      ones_v[pl.ds(i * L, L)] = jnp.ones((L,), jnp.float32)

  for k in range(IBUF):
    idx_load(k, k)
  for b in range(NBUF):
    idx_wait(b, b)
    pltpu.async_copy(h_hbm.at[src_i.at[b]], rows_v.at[b], gsem[b])
  plsc.subcore_barrier()

  def group(g, carry):
    for u in range(IBUF):
      j = g * IBUF + u
      b = u % NBUF
      s2 = (u + NBUF) % IBUF
      pltpu.make_async_copy(h_hbm.at[src_i.at[u]], rows_v.at[b],
                            gsem[b]).wait()
      pltpu.async_copy(rows_v.at[b], acc_sh.at[dst_i.at[u]], ssem[b],
                       add=True)
      if with_deg:
        pltpu.sync_copy(ones_v, deg_sh.at[dst_i.at[u]], add=True)
      pltpu.make_async_copy(rows_v.at[b], acc_sh.at[dst_i.at[u]],
                            ssem[b]).wait()

      @pl.when(j + IBUF < nch)
      def _():
        idx_load(j + IBUF, u)

      @pl.when(j + NBUF < nch)
      def _():
        idx_wait(j + NBUF, s2)
        pltpu.async_copy(h_hbm.at[src_i.at[s2]], rows_v.at[b], gsem[b])
    return carry

  lax.fori_loop(0, nch // IBUF, group, 0)
  plsc.subcore_barrier()

  pltpu.sync_copy(acc_sh.at[pl.ds(s * ZROWS, ZROWS)],
                  acc_out.at[c, pl.ds(s * ZROWS, ZROWS)])
  if with_deg:
    pltpu.sync_copy(deg_sh.at[pl.ds(s * ZROWS, ZROWS)],
                    deg_out.at[pl.ds(c * NPAD + s * ZROWS, ZROWS)])


def _make_sc_agg(with_deg):
  mesh = plsc.VectorSubcoreMesh(core_axis_name="c", subcore_axis_name="s",
                                num_cores=NC, num_subcores=NS)
  out_type = [jax.ShapeDtypeStruct((NC, NPAD, D), jnp.float32)]
  scratch = [
      pltpu.VMEM_SHARED((NPAD, D), jnp.float32),
  ]
  if with_deg:
    out_type.append(jax.ShapeDtypeStruct((NC * NPAD,), jnp.float32))
    scratch.append(pltpu.VMEM_SHARED((NPAD,), jnp.float32))
  scratch += [
      pltpu.VMEM((IBUF, CHUNK), jnp.int32),
      pltpu.VMEM((IBUF, CHUNK), jnp.int32),
      pltpu.VMEM((NBUF, CHUNK, D), jnp.float32),
  ]
  if with_deg:
    scratch.append(pltpu.VMEM((CHUNK,), jnp.float32))
  scratch += [pltpu.SemaphoreType.DMA] * (2 * NBUF + IBUF)
  return pl.kernel(
      functools.partial(_sc_agg_body, with_deg),
      out_type=tuple(out_type),
      mesh=mesh,
      scratch_types=tuple(scratch),
  )


_sc_agg_deg = _make_sc_agg(True)
_sc_agg = _make_sc_agg(False)


def _mm_body(x_ref, w_ref, b_ref, o_ref):
  o_ref[...] = (
      jnp.dot(x_ref[...], w_ref[...], preferred_element_type=jnp.float32)
      + b_ref[...]
  )


def _mm(x, w, b):
  m = x.shape[0]
  bm = 1000
  return pl.pallas_call(
      _mm_body,
      grid=(m // bm,),
      in_specs=[
          pl.BlockSpec((bm, D), lambda i: (i, 0)),
          pl.BlockSpec((D, D), lambda i: (0, 0)),
          pl.BlockSpec((1, D), lambda i: (0, 0)),
      ],
      out_specs=pl.BlockSpec((bm, D), lambda i: (i, 0)),
      out_shape=jax.ShapeDtypeStruct((m, D), jnp.float32),
  )(x, w, b)


def _scale_mm_body(p_ref, d_ref, w_ref, b_ref, o_ref):
  a = (p_ref[0] + p_ref[1]) * (d_ref[0] + d_ref[1])
  o_ref[...] = (
      jnp.dot(a, w_ref[...], preferred_element_type=jnp.float32) + b_ref[...]
  )


def _scale_mm(parts, degcol, w, b):
  m = parts.shape[1]
  bm = 1024
  return pl.pallas_call(
      _scale_mm_body,
      grid=(m // bm,),
      in_specs=[
          pl.BlockSpec((NC, bm, D), lambda i: (0, i, 0)),
          pl.BlockSpec((NC, bm, 1), lambda i: (0, i, 0)),
          pl.BlockSpec((D, D), lambda i: (0, 0)),
          pl.BlockSpec((1, D), lambda i: (0, 0)),
      ],
      out_specs=pl.BlockSpec((bm, D), lambda i: (i, 0)),
      out_shape=jax.ShapeDtypeStruct((m, D), jnp.float32),
  )(parts, degcol, w, b)


def kernel(x, edge_index, W1, b1, W2, b2, W3, b3):
  src = edge_index[0].astype(jnp.int32)
  dst = edge_index[1].astype(jnp.int32)
  e = src.shape[0]
  pad = NS * NCHUNK * CHUNK - e
  src3 = jnp.concatenate([src, jnp.zeros((pad,), jnp.int32)])
  dst3 = jnp.concatenate([dst, jnp.full((pad,), N_NODES, jnp.int32)])
  z2 = jnp.zeros((ZROWS, D), jnp.float32)
  z1 = jnp.zeros((ZROWS,), jnp.float32)
  b1r = b1.reshape(1, D)
  b2r = b2.reshape(1, D)
  n_cls = W3.shape[1]
  w3p = jnp.zeros((D, D), jnp.float32).at[:, :n_cls].set(W3)
  b3p = jnp.zeros((1, D), jnp.float32).at[0, :n_cls].set(b3)

  h1 = _mm(x, W1, b1r)
  acc1, degp = _sc_agg_deg(h1, src3, dst3, z2, z1)
  degcol = degp.reshape(NC, NPAD, 1)
  h2 = _scale_mm(acc1, degcol, W2, b2r)
  (acc2,) = _sc_agg(h2, src3, dst3, z2)
  outp = _scale_mm(acc2, degcol, w3p, b3p)
  return outp[:N_NODES, :n_cls]


# speedup vs baseline: 1.0584x; 1.0028x over previous
import functools

import jax
import jax.numpy as jnp
from jax import lax
from jax.experimental import pallas as pl
from jax.experimental.pallas import tpu as pltpu
from jax.experimental.pallas import tpu_sc as plsc

N_NODES = 10000
D = 128
NC, NS, L = 2, 16, 16
CHUNK = 64
NCHUNK = 320
NPAD = 10240
ZROWS = NPAD // NS
NBUF = 4
IBUF = 2 * NBUF
NCH0 = 304
NCH1 = NCHUNK - NCH0


def _sc_agg_body(with_deg, *refs):
  if with_deg:
    (h_hbm, src_hbm, dst_hbm, z2_hbm, z1_hbm, acc_out, deg_out,
     acc_sh, deg_sh, src_i, dst_i, rows_v, ones_v, *sems) = refs
  else:
    (h_hbm, src_hbm, dst_hbm, z2_hbm, acc_out,
     acc_sh, src_i, dst_i, rows_v, *sems) = refs
  gsem = sems[:NBUF]
  ssem = sems[NBUF:2 * NBUF]
  isem = sems[2 * NBUF:]
  c = lax.axis_index("c")
  s = lax.axis_index("s")
  nch = jnp.where(c == 0, NCH0, NCH1)
  base = jnp.where(c == 0, s * NCH0, NS * NCH0 + s * NCH1) * CHUNK

  def idx_load(k, slot):
    pltpu.async_copy(src_hbm.at[pl.ds(base + k * CHUNK, CHUNK)],
                     src_i.at[slot], isem[slot])
    pltpu.async_copy(dst_hbm.at[pl.ds(base + k * CHUNK, CHUNK)],
                     dst_i.at[slot], isem[slot])

  def idx_wait(k, slot):
    pltpu.make_async_copy(src_hbm.at[pl.ds(base + k * CHUNK, CHUNK)],
                          src_i.at[slot], isem[slot]).wait()
    pltpu.make_async_copy(dst_hbm.at[pl.ds(base + k * CHUNK, CHUNK)],
                          dst_i.at[slot], isem[slot]).wait()

  pltpu.sync_copy(z2_hbm, acc_sh.at[pl.ds(s * ZROWS, ZROWS)])
  if with_deg:
    pltpu.sync_copy(z1_hbm, deg_sh.at[pl.ds(s * ZROWS, ZROWS)])
    for i in range(CHUNK // L):
      ones_v[pl.ds(i * L, L)] = jnp.ones((L,), jnp.float32)

  for k in range(IBUF):
    idx_load(k, k)
  for b in range(NBUF):
    idx_wait(b, b)
    pltpu.async_copy(h_hbm.at[src_i.at[b]], rows_v.at[b], gsem[b])
  plsc.subcore_barrier()

  def group(g, carry):
    for u in range(IBUF):
      j = g * IBUF + u
      b = u % NBUF
      s2 = (u + NBUF) % IBUF
      pltpu.make_async_copy(h_hbm.at[src_i.at[u]], rows_v.at[b],
                            gsem[b]).wait()
      pltpu.async_copy(rows_v.at[b], acc_sh.at[dst_i.at[u]], ssem[b],
                       add=True)
      if with_deg:
        pltpu.sync_copy(ones_v, deg_sh.at[dst_i.at[u]], add=True)
      pltpu.make_async_copy(rows_v.at[b], acc_sh.at[dst_i.at[u]],
                            ssem[b]).wait()

      @pl.when(j + IBUF < nch)
      def _():
        idx_load(j + IBUF, u)

      @pl.when(j + NBUF < nch)
      def _():
        idx_wait(j + NBUF, s2)
        pltpu.async_copy(h_hbm.at[src_i.at[s2]], rows_v.at[b], gsem[b])
    return carry

  lax.fori_loop(0, nch // IBUF, group, 0)
  plsc.subcore_barrier()

  pltpu.sync_copy(acc_sh.at[pl.ds(s * ZROWS, ZROWS)],
                  acc_out.at[c, pl.ds(s * ZROWS, ZROWS)])
  if with_deg:
    pltpu.sync_copy(deg_sh.at[pl.ds(s * ZROWS, ZROWS)],
                    deg_out.at[pl.ds(c * NPAD + s * ZROWS, ZROWS)])


def _make_sc_agg(with_deg):
  mesh = plsc.VectorSubcoreMesh(core_axis_name="c", subcore_axis_name="s",
                                num_cores=NC, num_subcores=NS)
  out_type = [jax.ShapeDtypeStruct((NC, NPAD, D), jnp.float32)]
  scratch = [
      pltpu.VMEM_SHARED((NPAD, D), jnp.float32),
  ]
  if with_deg:
    out_type.append(jax.ShapeDtypeStruct((NC * NPAD,), jnp.float32))
    scratch.append(pltpu.VMEM_SHARED((NPAD,), jnp.float32))
  scratch += [
      pltpu.VMEM((IBUF, CHUNK), jnp.int32),
      pltpu.VMEM((IBUF, CHUNK), jnp.int32),
      pltpu.VMEM((NBUF, CHUNK, D), jnp.float32),
  ]
  if with_deg:
    scratch.append(pltpu.VMEM((CHUNK,), jnp.float32))
  scratch += [pltpu.SemaphoreType.DMA] * (2 * NBUF + IBUF)
  return pl.kernel(
      functools.partial(_sc_agg_body, with_deg),
      out_type=tuple(out_type),
      mesh=mesh,
      scratch_types=tuple(scratch),
  )


_sc_agg_deg = _make_sc_agg(True)
_sc_agg = _make_sc_agg(False)


def _mm_body(x_ref, w_ref, b_ref, o_ref):
  o_ref[...] = (
      jnp.dot(x_ref[...], w_ref[...], preferred_element_type=jnp.float32)
      + b_ref[...]
  )


def _mm(x, w, b):
  m = x.shape[0]
  bm = 1000
  return pl.pallas_call(
      _mm_body,
      grid=(m // bm,),
      in_specs=[
          pl.BlockSpec((bm, D), lambda i: (i, 0)),
          pl.BlockSpec((D, D), lambda i: (0, 0)),
          pl.BlockSpec((1, D), lambda i: (0, 0)),
      ],
      out_specs=pl.BlockSpec((bm, D), lambda i: (i, 0)),
      out_shape=jax.ShapeDtypeStruct((m, D), jnp.float32),
  )(x, w, b)


def _scale_mm_body(p_ref, d_ref, w_ref, b_ref, o_ref):
  a = (p_ref[0] + p_ref[1]) * (d_ref[0] + d_ref[1])
  o_ref[...] = (
      jnp.dot(a, w_ref[...], preferred_element_type=jnp.float32) + b_ref[...]
  )


def _scale_mm(parts, degcol, w, b):
  m = parts.shape[1]
  bm = 1024
  return pl.pallas_call(
      _scale_mm_body,
      grid=(m // bm,),
      in_specs=[
          pl.BlockSpec((NC, bm, D), lambda i: (0, i, 0)),
          pl.BlockSpec((NC, bm, 1), lambda i: (0, i, 0)),
          pl.BlockSpec((D, D), lambda i: (0, 0)),
          pl.BlockSpec((1, D), lambda i: (0, 0)),
      ],
      out_specs=pl.BlockSpec((bm, D), lambda i: (i, 0)),
      out_shape=jax.ShapeDtypeStruct((m, D), jnp.float32),
  )(parts, degcol, w, b)


def kernel(x, edge_index, W1, b1, W2, b2, W3, b3):
  src = edge_index[0].astype(jnp.int32)
  dst = edge_index[1].astype(jnp.int32)
  e = src.shape[0]
  pad = NS * NCHUNK * CHUNK - e
  src3 = jnp.concatenate([src, jnp.zeros((pad,), jnp.int32)])
  dst3 = jnp.concatenate([dst, jnp.full((pad,), N_NODES, jnp.int32)])
  z2 = jnp.zeros((ZROWS, D), jnp.float32)
  z1 = jnp.zeros((ZROWS,), jnp.float32)
  b1r = b1.reshape(1, D)
  b2r = b2.reshape(1, D)
  n_cls = W3.shape[1]
  w3p = jnp.zeros((D, D), jnp.float32).at[:, :n_cls].set(W3)
  b3p = jnp.zeros((1, D), jnp.float32).at[0, :n_cls].set(b3)

  h1 = _mm(x, W1, b1r)
  acc1, degp = _sc_agg_deg(h1, src3, dst3, z2, z1)
  degcol = degp.reshape(NC, NPAD, 1)
  h2 = _scale_mm(acc1, degcol, W2, b2r)
  (acc2,) = _sc_agg(h2, src3, dst3, z2)
  outp = _scale_mm(acc2, degcol, w3p, b3p)
  return outp[:N_NODES, :n_cls]


# speedup vs baseline: 1.0610x; 1.0025x over previous
import functools

import jax
import jax.numpy as jnp
from jax import lax
from jax.experimental import pallas as pl
from jax.experimental.pallas import tpu as pltpu
from jax.experimental.pallas import tpu_sc as plsc

N_NODES = 10000
D = 128
NC, NS, L = 2, 16, 16
CHUNK = 64
NCHUNK = 320
NPAD = 10240
ZROWS = NPAD // NS
NBUF = 4
IBUF = 2 * NBUF
NCH0 = 296
NCH1 = NCHUNK - NCH0


def _sc_agg_body(with_deg, *refs):
  if with_deg:
    (h_hbm, src_hbm, dst_hbm, z2_hbm, z1_hbm, acc_out, deg_out,
     acc_sh, deg_sh, src_i, dst_i, rows_v, ones_v, *sems) = refs
  else:
    (h_hbm, src_hbm, dst_hbm, z2_hbm, acc_out,
     acc_sh, src_i, dst_i, rows_v, *sems) = refs
  gsem = sems[:NBUF]
  ssem = sems[NBUF:2 * NBUF]
  isem = sems[2 * NBUF:]
  c = lax.axis_index("c")
  s = lax.axis_index("s")
  nch = jnp.where(c == 0, NCH0, NCH1)
  base = jnp.where(c == 0, s * NCH0, NS * NCH0 + s * NCH1) * CHUNK

  def idx_load(k, slot):
    pltpu.async_copy(src_hbm.at[pl.ds(base + k * CHUNK, CHUNK)],
                     src_i.at[slot], isem[slot])
    pltpu.async_copy(dst_hbm.at[pl.ds(base + k * CHUNK, CHUNK)],
                     dst_i.at[slot], isem[slot])

  def idx_wait(k, slot):
    pltpu.make_async_copy(src_hbm.at[pl.ds(base + k * CHUNK, CHUNK)],
                          src_i.at[slot], isem[slot]).wait()
    pltpu.make_async_copy(dst_hbm.at[pl.ds(base + k * CHUNK, CHUNK)],
                          dst_i.at[slot], isem[slot]).wait()

  pltpu.sync_copy(z2_hbm, acc_sh.at[pl.ds(s * ZROWS, ZROWS)])
  if with_deg:
    pltpu.sync_copy(z1_hbm, deg_sh.at[pl.ds(s * ZROWS, ZROWS)])
    for i in range(CHUNK // L):
      ones_v[pl.ds(i * L, L)] = jnp.ones((L,), jnp.float32)

  for k in range(IBUF):
    idx_load(k, k)
  for b in range(NBUF):
    idx_wait(b, b)
    pltpu.async_copy(h_hbm.at[src_i.at[b]], rows_v.at[b], gsem[b])
  plsc.subcore_barrier()

  def group(g, carry):
    for u in range(IBUF):
      j = g * IBUF + u
      b = u % NBUF
      s2 = (u + NBUF) % IBUF
      pltpu.make_async_copy(h_hbm.at[src_i.at[u]], rows_v.at[b],
                            gsem[b]).wait()
      pltpu.async_copy(rows_v.at[b], acc_sh.at[dst_i.at[u]], ssem[b],
                       add=True)
      if with_deg:
        pltpu.sync_copy(ones_v, deg_sh.at[dst_i.at[u]], add=True)
      pltpu.make_async_copy(rows_v.at[b], acc_sh.at[dst_i.at[u]],
                            ssem[b]).wait()

      @pl.when(j + IBUF < nch)
      def _():
        idx_load(j + IBUF, u)

      @pl.when(j + NBUF < nch)
      def _():
        idx_wait(j + NBUF, s2)
        pltpu.async_copy(h_hbm.at[src_i.at[s2]], rows_v.at[b], gsem[b])
    return carry

  lax.fori_loop(0, nch // IBUF, group, 0)
  plsc.subcore_barrier()

  pltpu.sync_copy(acc_sh.at[pl.ds(s * ZROWS, ZROWS)],
                  acc_out.at[c, pl.ds(s * ZROWS, ZROWS)])
  if with_deg:
    pltpu.sync_copy(deg_sh.at[pl.ds(s * ZROWS, ZROWS)],
                    deg_out.at[pl.ds(c * NPAD + s * ZROWS, ZROWS)])


def _make_sc_agg(with_deg):
  mesh = plsc.VectorSubcoreMesh(core_axis_name="c", subcore_axis_name="s",
                                num_cores=NC, num_subcores=NS)
  out_type = [jax.ShapeDtypeStruct((NC, NPAD, D), jnp.float32)]
  scratch = [
      pltpu.VMEM_SHARED((NPAD, D), jnp.float32),
  ]
  if with_deg:
    out_type.append(jax.ShapeDtypeStruct((NC * NPAD,), jnp.float32))
    scratch.append(pltpu.VMEM_SHARED((NPAD,), jnp.float32))
  scratch += [
      pltpu.VMEM((IBUF, CHUNK), jnp.int32),
      pltpu.VMEM((IBUF, CHUNK), jnp.int32),
      pltpu.VMEM((NBUF, CHUNK, D), jnp.float32),
  ]
  if with_deg:
    scratch.append(pltpu.VMEM((CHUNK,), jnp.float32))
  scratch += [pltpu.SemaphoreType.DMA] * (2 * NBUF + IBUF)
  return pl.kernel(
      functools.partial(_sc_agg_body, with_deg),
      out_type=tuple(out_type),
      mesh=mesh,
      scratch_types=tuple(scratch),
  )


_sc_agg_deg = _make_sc_agg(True)
_sc_agg = _make_sc_agg(False)


def _mm_body(x_ref, w_ref, b_ref, o_ref):
  o_ref[...] = (
      jnp.dot(x_ref[...], w_ref[...], preferred_element_type=jnp.float32)
      + b_ref[...]
  )


def _mm(x, w, b):
  m = x.shape[0]
  bm = 1000
  return pl.pallas_call(
      _mm_body,
      grid=(m // bm,),
      in_specs=[
          pl.BlockSpec((bm, D), lambda i: (i, 0)),
          pl.BlockSpec((D, D), lambda i: (0, 0)),
          pl.BlockSpec((1, D), lambda i: (0, 0)),
      ],
      out_specs=pl.BlockSpec((bm, D), lambda i: (i, 0)),
      out_shape=jax.ShapeDtypeStruct((m, D), jnp.float32),
  )(x, w, b)


def _scale_mm_body(p_ref, d_ref, w_ref, b_ref, o_ref):
  a = (p_ref[0] + p_ref[1]) * (d_ref[0] + d_ref[1])
  o_ref[...] = (
      jnp.dot(a, w_ref[...], preferred_element_type=jnp.float32) + b_ref[...]
  )


def _scale_mm(parts, degcol, w, b):
  m = parts.shape[1]
  bm = 1024
  return pl.pallas_call(
      _scale_mm_body,
      grid=(m // bm,),
      in_specs=[
          pl.BlockSpec((NC, bm, D), lambda i: (0, i, 0)),
          pl.BlockSpec((NC, bm, 1), lambda i: (0, i, 0)),
          pl.BlockSpec((D, D), lambda i: (0, 0)),
          pl.BlockSpec((1, D), lambda i: (0, 0)),
      ],
      out_specs=pl.BlockSpec((bm, D), lambda i: (i, 0)),
      out_shape=jax.ShapeDtypeStruct((m, D), jnp.float32),
  )(parts, degcol, w, b)


def kernel(x, edge_index, W1, b1, W2, b2, W3, b3):
  src = edge_index[0].astype(jnp.int32)
  dst = edge_index[1].astype(jnp.int32)
  e = src.shape[0]
  pad = NS * NCHUNK * CHUNK - e
  src3 = jnp.concatenate([src, jnp.zeros((pad,), jnp.int32)])
  dst3 = jnp.concatenate([dst, jnp.full((pad,), N_NODES, jnp.int32)])
  z2 = jnp.zeros((ZROWS, D), jnp.float32)
  z1 = jnp.zeros((ZROWS,), jnp.float32)
  b1r = b1.reshape(1, D)
  b2r = b2.reshape(1, D)
  n_cls = W3.shape[1]
  w3p = jnp.zeros((D, D), jnp.float32).at[:, :n_cls].set(W3)
  b3p = jnp.zeros((1, D), jnp.float32).at[0, :n_cls].set(b3)

  h1 = _mm(x, W1, b1r)
  acc1, degp = _sc_agg_deg(h1, src3, dst3, z2, z1)
  degcol = degp.reshape(NC, NPAD, 1)
  h2 = _scale_mm(acc1, degcol, W2, b2r)
  (acc2,) = _sc_agg(h2, src3, dst3, z2)
  outp = _scale_mm(acc2, degcol, w3p, b3p)
  return outp[:N_NODES, :n_cls]
